# baseline (speedup 1.0000x reference)
import functools

import jax
import jax.numpy as jnp
from jax import lax
from jax.experimental import pallas as pl
from jax.experimental.pallas import tpu as pltpu
from jax.experimental.pallas import tpu_sc as plsc

N_NODES = 10000
D = 128
HD = 64
N_EDGES = 320000

NC = 2
NS = 16
NW = NC * NS

NPAD = 10240
ROWS_PER_TILE = NPAD // NS
PAD_ROWS = NPAD - N_NODES

CHUNK = 128
EPAD = 327680
NCH_TOT = EPAD // CHUNK
K1_NCH = NCH_TOT // NW
K3_NCH = NCH_TOT // NS

WIN = 40
NBUF = 6

_MESH = plsc.VectorSubcoreMesh(core_axis_name="c", subcore_axis_name="s")


@functools.partial(
    pl.kernel,
    mesh=_MESH,
    out_type=[
        jax.ShapeDtypeStruct((NC, NPAD), jnp.float32),
        jax.ShapeDtypeStruct((NC, NPAD), jnp.float32),
    ],
    scratch_types=[
        pltpu.VMEM((K1_NCH, CHUNK), jnp.int32),
        pltpu.VMEM((K1_NCH, CHUNK), jnp.int32),
        pltpu.VMEM((CHUNK,), jnp.float32),
        pltpu.VMEM_SHARED((NPAD,), jnp.float32),
        pltpu.VMEM_SHARED((NPAD,), jnp.float32),
        pltpu.SemaphoreType.DMA,
    ],
)
def _sc_degrees(e2d_hbm, z1d_hbm, dout_hbm, din_hbm,
                sidx, didx, ones, dout_sh, din_sh, ssem):
    c = lax.axis_index("c")
    s = lax.axis_index("s")
    wid = c * NS + s

    def fill16(i, _):
        ones[pl.ds(i * 16, 16)] = jnp.full((16,), 1.0, jnp.float32)
        return 0
    lax.fori_loop(0, CHUNK // 16, fill16, 0)

    stripe = pl.ds(s * ROWS_PER_TILE, ROWS_PER_TILE)
    pltpu.sync_copy(z1d_hbm.at[stripe], dout_sh.at[stripe])
    pltpu.sync_copy(z1d_hbm.at[stripe], din_sh.at[stripe])

    pltpu.sync_copy(e2d_hbm.at[0, pl.ds(wid * K1_NCH, K1_NCH)], sidx)
    pltpu.sync_copy(e2d_hbm.at[1, pl.ds(wid * K1_NCH, K1_NCH)], didx)

    plsc.subcore_barrier()

    def fire(j, _):
        pltpu.async_copy(ones, dout_sh.at[sidx.at[j]], ssem, add=True)
        pltpu.async_copy(ones, din_sh.at[didx.at[j]], ssem, add=True)
        return 0
    lax.fori_loop(0, K1_NCH, fire, 0)

    def drain(j, _):
        pltpu.make_async_copy(ones, dout_sh.at[sidx.at[0]], ssem).wait()
        pltpu.make_async_copy(ones, din_sh.at[didx.at[0]], ssem).wait()
        return 0
    lax.fori_loop(0, K1_NCH, drain, 0)

    plsc.subcore_barrier()

    pltpu.sync_copy(dout_sh.at[stripe], dout_hbm.at[c, stripe])
    pltpu.sync_copy(din_sh.at[stripe], din_hbm.at[c, stripe])


@functools.partial(
    pl.kernel,
    mesh=_MESH,
    compiler_params=pltpu.CompilerParams(use_tc_tiling_on_sc=False),
    out_type=[
        jax.ShapeDtypeStruct((NC, NPAD, HD), jnp.float32),
        jax.ShapeDtypeStruct((NC, NPAD), jnp.float32),
    ],
    scratch_types=[
        pltpu.VMEM((WIN, CHUNK), jnp.int32),
        pltpu.VMEM((WIN, CHUNK), jnp.int32),
        pltpu.VMEM((NBUF, CHUNK, HD), jnp.float32),
        pltpu.VMEM((WIN, CHUNK), jnp.float32),
        pltpu.VMEM_SHARED((NPAD, HD), jnp.float32),
        pltpu.VMEM_SHARED((NPAD,), jnp.float32),
        pltpu.SemaphoreType.DMA,
        pltpu.SemaphoreType.DMA,
        pltpu.SemaphoreType.DMA,
        pltpu.SemaphoreType.DMA,
        pltpu.SemaphoreType.DMA,
        pltpu.SemaphoreType.DMA,
        pltpu.SemaphoreType.DMA,
        pltpu.SemaphoreType.DMA,
        pltpu.SemaphoreType.DMA,
        pltpu.SemaphoreType.DMA,
        pltpu.SemaphoreType.DMA,
        pltpu.SemaphoreType.DMA,
        pltpu.SemaphoreType.DMA,
        pltpu.SemaphoreType.DMA,
    ],
)
def _sc_scatter(hw2_hbm, e2d_hbm, rin_hbm, z2d_hbm, z1d_hbm,
                agg_hbm, c_hbm,
                sidx, didx, rows, rvals, agg_sh, c_sh,
                gs0, gs1, gs2, gs3, gs4, gs5,
                ss0, ss1, ss2, ss3, ss4, ss5, rsem, csem):
    cfh = lax.axis_index("c")
    t = lax.axis_index("s")
    gs = (gs0, gs1, gs2, gs3, gs4, gs5)
    ss = (ss0, ss1, ss2, ss3, ss4, ss5)

    stripe = pl.ds(t * ROWS_PER_TILE, ROWS_PER_TILE)
    pltpu.sync_copy(z2d_hbm.at[stripe], agg_sh.at[stripe])
    pltpu.sync_copy(z1d_hbm.at[stripe], c_sh.at[stripe])

    plsc.subcore_barrier()

    base = t * K3_NCH
    hw2_half = hw2_hbm.at[cfh]

    for p in range(K3_NCH // WIN):
        pltpu.sync_copy(e2d_hbm.at[0, pl.ds(base + p * WIN, WIN)], sidx)
        pltpu.sync_copy(e2d_hbm.at[1, pl.ds(base + p * WIN, WIN)], didx)

        for q in range(NBUF - 2):
            pltpu.async_copy(hw2_half.at[sidx.at[q]], rows.at[q], gs[q])

        def slot6(i, _):
            for b in range(NBUF):
                m = i * NBUF + b

                @pl.when(m < WIN)
                def _():
                    pltpu.make_async_copy(hw2_half.at[sidx.at[0]],
                                          rows.at[b], gs[b]).wait()
                    pltpu.async_copy(rows.at[b], agg_sh.at[didx.at[m]],
                                     ss[b], add=True)

                    @pl.when(m + NBUF - 2 < WIN)
                    def _():
                        b2 = (b + NBUF - 2) % NBUF

                        @pl.when(m >= 2)
                        def _():
                            pltpu.make_async_copy(
                                rows.at[b2], agg_sh.at[pl.ds(0, CHUNK)],
                                ss[b2]).wait()
                        pltpu.async_copy(hw2_half.at[sidx.at[m + NBUF - 2]],
                                         rows.at[b2], gs[b2])
            return 0
        lax.fori_loop(0, (WIN + NBUF - 1) // NBUF, slot6, 0)

        for b in range(NBUF):
            pltpu.make_async_copy(rows.at[b], agg_sh.at[pl.ds(0, CHUNK)],
                                  ss[b]).wait()

        do_c = (p % 2) == cfh

        @pl.when(do_c)
        def _():
            def fire_rin(m, _):
                pltpu.async_copy(rin_hbm.at[didx.at[m]], rvals.at[m], rsem)
                return 0
            lax.fori_loop(0, WIN, fire_rin, 0)

            def drain_rin(m, _):
                pltpu.make_async_copy(rin_hbm.at[didx.at[0]], rvals.at[0],
                                      rsem).wait()
                return 0
            lax.fori_loop(0, WIN, drain_rin, 0)

            def fire_c(m, _):
                pltpu.async_copy(rvals.at[m], c_sh.at[sidx.at[m]], csem,
                                 add=True)
                return 0
            lax.fori_loop(0, WIN, fire_c, 0)

            def drain_c(m, _):
                pltpu.make_async_copy(rvals.at[0], c_sh.at[sidx.at[0]],
                                      csem).wait()
                return 0
            lax.fori_loop(0, WIN, drain_c, 0)

    plsc.subcore_barrier()

    pltpu.sync_copy(agg_sh.at[stripe], agg_hbm.at[cfh, stripe])
    pltpu.sync_copy(c_sh.at[stripe], c_hbm.at[cfh, stripe])


def _fuse_body(x_ref, w_ref, do_ref, di_ref, hw2_ref, ro_ref, ri_ref):
    do = do_ref[0] + do_ref[1]
    di = di_ref[0] + di_ref[1]
    ro = lax.rsqrt(jnp.maximum(do, 1.0))
    ri = lax.rsqrt(jnp.maximum(di, 1.0))
    res = jnp.dot(jnp.sin(x_ref[...]) * ro, w_ref[...],
                  preferred_element_type=jnp.float32)
    hw2_ref[0, :, :] = res[:, :HD]
    hw2_ref[1, :, :] = res[:, HD:]
    ro_ref[...] = ro
    ri_ref[...] = ri


def _tc_fused(xpad, W1, doutp, dinp):
    blk = 512
    return pl.pallas_call(
        _fuse_body,
        grid=(NPAD // blk,),
        in_specs=[
            pl.BlockSpec((blk, D), lambda i: (i, 0)),
            pl.BlockSpec((D, D), lambda i: (0, 0)),
            pl.BlockSpec((NC, blk, 1), lambda i: (0, i, 0)),
            pl.BlockSpec((NC, blk, 1), lambda i: (0, i, 0)),
        ],
        out_specs=[
            pl.BlockSpec((NC, blk, HD), lambda i: (0, i, 0)),
            pl.BlockSpec((blk, 1), lambda i: (i, 0)),
            pl.BlockSpec((blk, 1), lambda i: (i, 0)),
        ],
        out_shape=[
            jax.ShapeDtypeStruct((NC, NPAD, HD), jnp.float32),
            jax.ShapeDtypeStruct((NPAD, 1), jnp.float32),
            jax.ShapeDtypeStruct((NPAD, 1), jnp.float32),
        ],
    )(xpad, W1, doutp, dinp)


def _final_body(agg_ref, ri_ref, b1_ref, w2_ref, ro_ref, c_ref, b2_ref,
                o_ref):
    pid = pl.program_id(0)
    blk = agg_ref.shape[1]
    agg = jnp.concatenate([agg_ref[0], agg_ref[1]], axis=1)
    out1 = agg * ri_ref[...] + b1_ref[...]
    t = jnp.tanh(out1)
    w2s = jnp.sum(w2_ref[...], axis=1, keepdims=True)
    sblk = jnp.dot(t, w2s, preferred_element_type=jnp.float32)
    cc = c_ref[0] + c_ref[1]
    rowid = pid * blk + lax.broadcasted_iota(jnp.int32, (blk, 1), 0)
    valid = jnp.where(rowid < N_NODES, 1.0, 0.0)
    part = jnp.sum(ro_ref[...] * sblk * cc * valid, keepdims=True)

    @pl.when(pid == 0)
    def _():
        o_ref[...] = jnp.float32(N_NODES) * jnp.sum(b2_ref[...],
                                                    keepdims=True)

    o_ref[...] += part


def _tc_final(aggp, rin, b1, W2, rout, cp, b2):
    blk = 512
    return pl.pallas_call(
        _final_body,
        grid=(NPAD // blk,),
        in_specs=[
            pl.BlockSpec((NC, blk, HD), lambda i: (0, i, 0)),
            pl.BlockSpec((blk, 1), lambda i: (i, 0)),
            pl.BlockSpec((1, D), lambda i: (0, 0)),
            pl.BlockSpec((D, D), lambda i: (0, 0)),
            pl.BlockSpec((blk, 1), lambda i: (i, 0)),
            pl.BlockSpec((NC, blk, 1), lambda i: (0, i, 0)),
            pl.BlockSpec((1, D), lambda i: (0, 0)),
        ],
        out_specs=pl.BlockSpec((1, 1), lambda i: (0, 0)),
        out_shape=jax.ShapeDtypeStruct((1, 1), jnp.float32),
    )(aggp, rin, b1, W2, rout, cp, b2)


def kernel(x, edge_index, W1, b1, W2, b2):
    ei = edge_index.astype(jnp.int32)

    npad_e = EPAD - N_EDGES
    pad_idx = N_NODES + (jnp.arange(npad_e, dtype=jnp.int32) % PAD_ROWS)
    src_p = jnp.concatenate([ei[0], pad_idx])
    dst_p = jnp.concatenate([ei[1], pad_idx])
    e2d = jnp.stack([src_p, dst_p]).reshape(2, NCH_TOT, CHUNK)

    z2d = jnp.zeros((NPAD, HD), jnp.float32)
    z1d = jnp.zeros((NPAD,), jnp.float32)

    doutp, dinp = _sc_degrees(e2d, z1d)

    xpad = jnp.pad(x, ((0, NPAD - N_NODES), (0, 0)))
    hw2, rout, rin = _tc_fused(
        xpad, W1,
        doutp.reshape(NC, NPAD, 1), dinp.reshape(NC, NPAD, 1))

    aggp, cp = _sc_scatter(hw2, e2d, rin.reshape(NPAD), z2d, z1d)

    out = _tc_final(aggp, rin, b1.reshape(1, D), W2, rout,
                    cp.reshape(NC, NPAD, 1), b2.reshape(1, D))
    return out


# speedup vs baseline: 1.0241x; 1.0241x over previous
import functools

import jax
import jax.numpy as jnp
from jax import lax
from jax.experimental import pallas as pl
from jax.experimental.pallas import tpu as pltpu
from jax.experimental.pallas import tpu_sc as plsc

N_NODES = 10000
D = 128
N_EDGES = 320000

NC = 2
NS = 16
NW = NC * NS

NPAD = 10240
ROWS_PER_TILE = NPAD // NS
PAD_ROWS = NPAD - N_NODES

CHUNK = 128
EPAD = 327680
EDGES_PER_W = EPAD // NW
NCHUNK = EDGES_PER_W // CHUNK

_MESH = plsc.VectorSubcoreMesh(core_axis_name="c", subcore_axis_name="s")


def _worker_ids():
    c = lax.axis_index("c")
    s = lax.axis_index("s")
    return c, s, c * NS + s


@functools.partial(
    pl.kernel,
    mesh=_MESH,
    out_type=[
        jax.ShapeDtypeStruct((NC, NPAD), jnp.float32),
        jax.ShapeDtypeStruct((NC, NPAD), jnp.float32),
    ],
    scratch_types=[
        pltpu.VMEM((NCHUNK, CHUNK), jnp.int32),
        pltpu.VMEM((NCHUNK, CHUNK), jnp.int32),
        pltpu.VMEM((CHUNK,), jnp.float32),
        pltpu.VMEM_SHARED((NPAD,), jnp.float32),
        pltpu.VMEM_SHARED((NPAD,), jnp.float32),
        pltpu.SemaphoreType.DMA,
    ],
)
def _sc_degrees(src_hbm, dst_hbm, z1d_hbm, dout_hbm, din_hbm,
                sidx, didx, ones, dout_sh, din_sh, ssem):
    c, s, wid = _worker_ids()

    def fill16(i, _):
        ones[pl.ds(i * 16, 16)] = jnp.full((16,), 1.0, jnp.float32)
        return 0
    lax.fori_loop(0, CHUNK // 16, fill16, 0)

    stripe = pl.ds(s * ROWS_PER_TILE, ROWS_PER_TILE)
    pltpu.sync_copy(z1d_hbm.at[stripe], dout_sh.at[stripe])
    pltpu.sync_copy(z1d_hbm.at[stripe], din_sh.at[stripe])

    pltpu.sync_copy(src_hbm.at[pl.ds(wid * NCHUNK, NCHUNK)], sidx)
    pltpu.sync_copy(dst_hbm.at[pl.ds(wid * NCHUNK, NCHUNK)], didx)

    plsc.subcore_barrier()

    def round_(r, _):
        for b in range(8):
            j = r * 8 + b
            pltpu.async_copy(ones, dout_sh.at[sidx.at[j]], ssem, add=True)
            pltpu.async_copy(ones, din_sh.at[didx.at[j]], ssem, add=True)
        for b in range(8):
            j = r * 8 + b
            pltpu.make_async_copy(ones, dout_sh.at[sidx.at[j]], ssem).wait()
            pltpu.make_async_copy(ones, din_sh.at[didx.at[j]], ssem).wait()
        return 0
    lax.fori_loop(0, NCHUNK // 8, round_, 0)

    plsc.subcore_barrier()

    pltpu.sync_copy(dout_sh.at[stripe], dout_hbm.at[c, stripe])
    pltpu.sync_copy(din_sh.at[stripe], din_hbm.at[c, stripe])


@functools.partial(
    pl.kernel,
    mesh=_MESH,
    out_type=[
        jax.ShapeDtypeStruct((NC, NPAD, D), jnp.float32),
        jax.ShapeDtypeStruct((NC, NPAD), jnp.float32),
    ],
    scratch_types=[
        pltpu.VMEM((NCHUNK // 2, CHUNK), jnp.int32),
        pltpu.VMEM((NCHUNK // 2, CHUNK), jnp.int32),
        pltpu.VMEM((2, CHUNK, D), jnp.float32),
        pltpu.VMEM((8, CHUNK), jnp.float32),
        pltpu.VMEM_SHARED((NPAD, D), jnp.float32),
        pltpu.VMEM_SHARED((NPAD,), jnp.float32),
        pltpu.SemaphoreType.DMA,
        pltpu.SemaphoreType.DMA,
        pltpu.SemaphoreType.DMA,
        pltpu.SemaphoreType.DMA,
    ],
)
def _sc_scatter(hw2_hbm, src_hbm, dst_hbm, rin_hbm, z2d_hbm, z1d_hbm,
                agg_hbm, c_hbm,
                sidx, didx, rows, rvals, agg_sh, c_sh,
                gs0, gs1, rsem, csem):
    c, s, wid = _worker_ids()
    gs = (gs0, gs1)
    HALF = NCHUNK // 2

    stripe = pl.ds(s * ROWS_PER_TILE, ROWS_PER_TILE)
    pltpu.sync_copy(z2d_hbm.at[stripe], agg_sh.at[stripe])
    pltpu.sync_copy(z1d_hbm.at[stripe], c_sh.at[stripe])

    plsc.subcore_barrier()

    for h in range(2):
        pltpu.sync_copy(src_hbm.at[pl.ds(wid * NCHUNK + h * HALF, HALF)], sidx)
        pltpu.sync_copy(dst_hbm.at[pl.ds(wid * NCHUNK + h * HALF, HALF)], didx)

        for b in range(2):
            pltpu.async_copy(hw2_hbm.at[sidx.at[b]], rows.at[b], gs[b])

        def main(i, _):
            for b in range(2):
                j = i * 2 + b
                pltpu.make_async_copy(hw2_hbm.at[sidx.at[j]], rows.at[b],
                                      gs[b]).wait()
                pltpu.sync_copy(rows.at[b], agg_sh.at[didx.at[j]], add=True)

                @pl.when(j + 2 < HALF)
                def _():
                    pltpu.async_copy(hw2_hbm.at[sidx.at[j + 2]], rows.at[b],
                                     gs[b])
            return 0
        lax.fori_loop(0, HALF // 2, main, 0)

        def cround(r, _):
            for b in range(8):
                pltpu.async_copy(rin_hbm.at[didx.at[r * 8 + b]], rvals.at[b],
                                 rsem)
            for b in range(8):
                pltpu.make_async_copy(rin_hbm.at[didx.at[r * 8 + b]],
                                      rvals.at[b], rsem).wait()
            for b in range(8):
                pltpu.async_copy(rvals.at[b], c_sh.at[sidx.at[r * 8 + b]],
                                 csem, add=True)
            for b in range(8):
                pltpu.make_async_copy(rvals.at[b], c_sh.at[sidx.at[r * 8 + b]],
                                      csem).wait()
            return 0
        lax.fori_loop(0, HALF // 8, cround, 0)

    plsc.subcore_barrier()

    pltpu.sync_copy(agg_sh.at[stripe], agg_hbm.at[c, stripe])
    pltpu.sync_copy(c_sh.at[stripe], c_hbm.at[c, stripe])


def _fuse_body(x_ref, w_ref, do_ref, di_ref, hw2_ref, ro_ref, ri_ref):
    do = do_ref[0] + do_ref[1]
    di = di_ref[0] + di_ref[1]
    ro = lax.rsqrt(jnp.maximum(do, 1.0))
    ri = lax.rsqrt(jnp.maximum(di, 1.0))
    hw2_ref[...] = jnp.dot(jnp.sin(x_ref[...]) * ro, w_ref[...],
                           preferred_element_type=jnp.float32)
    ro_ref[...] = ro
    ri_ref[...] = ri


def _tc_fused(xpad, W1, doutp, dinp):
    blk = 512
    return pl.pallas_call(
        _fuse_body,
        grid=(NPAD // blk,),
        in_specs=[
            pl.BlockSpec((blk, D), lambda i: (i, 0)),
            pl.BlockSpec((D, D), lambda i: (0, 0)),
            pl.BlockSpec((NC, blk, 1), lambda i: (0, i, 0)),
            pl.BlockSpec((NC, blk, 1), lambda i: (0, i, 0)),
        ],
        out_specs=[
            pl.BlockSpec((blk, D), lambda i: (i, 0)),
            pl.BlockSpec((blk, 1), lambda i: (i, 0)),
            pl.BlockSpec((blk, 1), lambda i: (i, 0)),
        ],
        out_shape=[
            jax.ShapeDtypeStruct((NPAD, D), jnp.float32),
            jax.ShapeDtypeStruct((NPAD, 1), jnp.float32),
            jax.ShapeDtypeStruct((NPAD, 1), jnp.float32),
        ],
    )(xpad, W1, doutp, dinp)


def _final_body(agg_ref, ri_ref, b1_ref, w2_ref, ro_ref, c_ref, b2_ref, o_ref):
    pid = pl.program_id(0)
    blk = agg_ref.shape[1]
    agg = agg_ref[0] + agg_ref[1]
    out1 = agg * ri_ref[...] + b1_ref[...]
    t = jnp.tanh(out1)
    w2s = jnp.sum(w2_ref[...], axis=1, keepdims=True)
    sblk = jnp.dot(t, w2s, preferred_element_type=jnp.float32)
    cc = c_ref[0] + c_ref[1]
    rowid = pid * blk + lax.broadcasted_iota(jnp.int32, (blk, 1), 0)
    valid = jnp.where(rowid < N_NODES, 1.0, 0.0)
    part = jnp.sum(ro_ref[...] * sblk * cc * valid, keepdims=True)

    @pl.when(pid == 0)
    def _():
        o_ref[...] = jnp.float32(N_NODES) * jnp.sum(b2_ref[...], keepdims=True)

    o_ref[...] += part


def _tc_final(aggp, rin, b1, W2, rout, cp, b2):
    blk = 512
    return pl.pallas_call(
        _final_body,
        grid=(NPAD // blk,),
        in_specs=[
            pl.BlockSpec((NC, blk, D), lambda i: (0, i, 0)),
            pl.BlockSpec((blk, 1), lambda i: (i, 0)),
            pl.BlockSpec((1, D), lambda i: (0, 0)),
            pl.BlockSpec((D, D), lambda i: (0, 0)),
            pl.BlockSpec((blk, 1), lambda i: (i, 0)),
            pl.BlockSpec((NC, blk, 1), lambda i: (0, i, 0)),
            pl.BlockSpec((1, D), lambda i: (0, 0)),
        ],
        out_specs=pl.BlockSpec((1, 1), lambda i: (0, 0)),
        out_shape=jax.ShapeDtypeStruct((1, 1), jnp.float32),
    )(aggp, rin, b1, W2, rout, cp, b2)


def kernel(x, edge_index, W1, b1, W2, b2):
    ei = edge_index.astype(jnp.int32)
    src = ei[0]
    dst = ei[1]

    npad_e = EPAD - N_EDGES
    pad_idx = N_NODES + (jnp.arange(npad_e, dtype=jnp.int32) % PAD_ROWS)
    src_p = jnp.concatenate([src, pad_idx]).reshape(EPAD // CHUNK, CHUNK)
    dst_p = jnp.concatenate([dst, pad_idx]).reshape(EPAD // CHUNK, CHUNK)

    z2d = jnp.zeros((NPAD, D), jnp.float32)
    z1d = jnp.zeros((NPAD,), jnp.float32)

    doutp, dinp = _sc_degrees(src_p, dst_p, z1d)

    xpad = jnp.pad(x, ((0, NPAD - N_NODES), (0, 0)))
    hw2, rout, rin = _tc_fused(
        xpad, W1, doutp.reshape(NC, NPAD, 1), dinp.reshape(NC, NPAD, 1))

    aggp, cp = _sc_scatter(hw2, src_p, dst_p, rin.reshape(NPAD), z2d, z1d)

    out = _tc_final(aggp, rin, b1.reshape(1, D), W2, rout,
                    cp.reshape(NC, NPAD, 1), b2.reshape(1, D))
    return out


# speedup vs baseline: 1.0710x; 1.0459x over previous
import functools

import jax
import jax.numpy as jnp
from jax import lax
from jax.experimental import pallas as pl
from jax.experimental.pallas import tpu as pltpu
from jax.experimental.pallas import tpu_sc as plsc

N_NODES = 10000
D = 128
N_EDGES = 320000

NC = 2
NS = 16
NW = NC * NS

NPAD = 10240
ROWS_PER_TILE = NPAD // NS
PAD_ROWS = NPAD - N_NODES

CHUNK = 128
EPAD = 327680
EDGES_PER_W = EPAD // NW
NCHUNK = EDGES_PER_W // CHUNK

_MESH = plsc.VectorSubcoreMesh(core_axis_name="c", subcore_axis_name="s")


def _worker_ids():
    c = lax.axis_index("c")
    s = lax.axis_index("s")
    return c, s, c * NS + s


@functools.partial(
    pl.kernel,
    mesh=_MESH,
    out_type=[
        jax.ShapeDtypeStruct((NC, NPAD), jnp.float32),
        jax.ShapeDtypeStruct((NC, NPAD), jnp.float32),
    ],
    scratch_types=[
        pltpu.VMEM((NCHUNK, CHUNK), jnp.int32),
        pltpu.VMEM((NCHUNK, CHUNK), jnp.int32),
        pltpu.VMEM((CHUNK,), jnp.float32),
        pltpu.VMEM_SHARED((NPAD,), jnp.float32),
        pltpu.VMEM_SHARED((NPAD,), jnp.float32),
        pltpu.SemaphoreType.DMA,
    ],
)
def _sc_degrees(src_hbm, dst_hbm, z1d_hbm, dout_hbm, din_hbm,
                sidx, didx, ones, dout_sh, din_sh, ssem):
    c, s, wid = _worker_ids()

    def fill16(i, _):
        ones[pl.ds(i * 16, 16)] = jnp.full((16,), 1.0, jnp.float32)
        return 0
    lax.fori_loop(0, CHUNK // 16, fill16, 0)

    stripe = pl.ds(s * ROWS_PER_TILE, ROWS_PER_TILE)
    pltpu.sync_copy(z1d_hbm.at[stripe], dout_sh.at[stripe])
    pltpu.sync_copy(z1d_hbm.at[stripe], din_sh.at[stripe])

    pltpu.sync_copy(src_hbm.at[pl.ds(wid * NCHUNK, NCHUNK)], sidx)
    pltpu.sync_copy(dst_hbm.at[pl.ds(wid * NCHUNK, NCHUNK)], didx)

    plsc.subcore_barrier()

    def round_(r, _):
        for b in range(8):
            j = r * 8 + b
            pltpu.async_copy(ones, dout_sh.at[sidx.at[j]], ssem, add=True)
            pltpu.async_copy(ones, din_sh.at[didx.at[j]], ssem, add=True)
        for b in range(8):
            j = r * 8 + b
            pltpu.make_async_copy(ones, dout_sh.at[sidx.at[j]], ssem).wait()
            pltpu.make_async_copy(ones, din_sh.at[didx.at[j]], ssem).wait()
        return 0
    lax.fori_loop(0, NCHUNK // 8, round_, 0)

    plsc.subcore_barrier()

    pltpu.sync_copy(dout_sh.at[stripe], dout_hbm.at[c, stripe])
    pltpu.sync_copy(din_sh.at[stripe], din_hbm.at[c, stripe])


@functools.partial(
    pl.kernel,
    mesh=_MESH,
    out_type=[
        jax.ShapeDtypeStruct((NC, NPAD, D), jnp.float32),
        jax.ShapeDtypeStruct((NC, NPAD), jnp.float32),
    ],
    scratch_types=[
        pltpu.VMEM((NCHUNK // 2, CHUNK), jnp.int32),
        pltpu.VMEM((NCHUNK // 2, CHUNK), jnp.int32),
        pltpu.VMEM((2, CHUNK, D), jnp.float32),
        pltpu.VMEM((8, CHUNK), jnp.float32),
        pltpu.VMEM_SHARED((NPAD, D), jnp.float32),
        pltpu.VMEM_SHARED((NPAD,), jnp.float32),
        pltpu.SemaphoreType.DMA,
        pltpu.SemaphoreType.DMA,
        pltpu.SemaphoreType.DMA,
        pltpu.SemaphoreType.DMA,
    ],
)
def _sc_scatter(hw2_hbm, src_hbm, dst_hbm, rin_hbm, z2d_hbm, z1d_hbm,
                agg_hbm, c_hbm,
                sidx, didx, rows, rvals, agg_sh, c_sh,
                gs0, gs1, rsem, csem):
    c, s, wid = _worker_ids()
    gs = (gs0, gs1)
    HALF = NCHUNK // 2

    stripe = pl.ds(s * ROWS_PER_TILE, ROWS_PER_TILE)
    pltpu.sync_copy(z2d_hbm.at[stripe], agg_sh.at[stripe])
    pltpu.sync_copy(z1d_hbm.at[stripe], c_sh.at[stripe])

    plsc.subcore_barrier()

    for h in range(2):
        pltpu.sync_copy(src_hbm.at[pl.ds(wid * NCHUNK + h * HALF, HALF)], sidx)
        pltpu.sync_copy(dst_hbm.at[pl.ds(wid * NCHUNK + h * HALF, HALF)], didx)

        for b in range(2):
            pltpu.async_copy(hw2_hbm.at[sidx.at[b]], rows.at[b], gs[b])

        def main(i, _):
            for b in range(2):
                j = i * 2 + b
                pltpu.make_async_copy(hw2_hbm.at[sidx.at[j]], rows.at[b],
                                      gs[b]).wait()
                pltpu.sync_copy(rows.at[b], agg_sh.at[didx.at[j]], add=True)

                @pl.when(j + 2 < HALF)
                def _():
                    pltpu.async_copy(hw2_hbm.at[sidx.at[j + 2]], rows.at[b],
                                     gs[b])
            return 0
        lax.fori_loop(0, HALF // 2, main, 0)

        def cround(r, _):
            for b in range(8):
                pltpu.async_copy(rin_hbm.at[didx.at[r * 8 + b]], rvals.at[b],
                                 rsem)
            for b in range(8):
                pltpu.make_async_copy(rin_hbm.at[didx.at[r * 8 + b]],
                                      rvals.at[b], rsem).wait()
            for b in range(8):
                pltpu.async_copy(rvals.at[b], c_sh.at[sidx.at[r * 8 + b]],
                                 csem, add=True)
            for b in range(8):
                pltpu.make_async_copy(rvals.at[b], c_sh.at[sidx.at[r * 8 + b]],
                                      csem).wait()
            return 0
        lax.fori_loop(0, HALF // 8, cround, 0)

    plsc.subcore_barrier()

    pltpu.sync_copy(agg_sh.at[stripe], agg_hbm.at[c, stripe])
    pltpu.sync_copy(c_sh.at[stripe], c_hbm.at[c, stripe])


def _fuse_body(x_ref, w_ref, do_ref, di_ref, hw2_ref, ro_ref, ri_ref):
    do = do_ref[0] + do_ref[1]
    di = di_ref[0] + di_ref[1]
    ro = lax.rsqrt(jnp.maximum(do, 1.0))
    ri = lax.rsqrt(jnp.maximum(di, 1.0))
    hw2_ref[...] = jnp.dot(jnp.sin(x_ref[...]) * ro, w_ref[...],
                           preferred_element_type=jnp.float32)
    ro_ref[...] = ro
    ri_ref[...] = ri


def _tc_fused(xpad, W1, doutp, dinp):
    blk = 1024
    return pl.pallas_call(
        _fuse_body,
        grid=(NPAD // blk,),
        in_specs=[
            pl.BlockSpec((blk, D), lambda i: (i, 0)),
            pl.BlockSpec((D, D), lambda i: (0, 0)),
            pl.BlockSpec((NC, blk, 1), lambda i: (0, i, 0)),
            pl.BlockSpec((NC, blk, 1), lambda i: (0, i, 0)),
        ],
        out_specs=[
            pl.BlockSpec((blk, D), lambda i: (i, 0)),
            pl.BlockSpec((blk, 1), lambda i: (i, 0)),
            pl.BlockSpec((blk, 1), lambda i: (i, 0)),
        ],
        out_shape=[
            jax.ShapeDtypeStruct((NPAD, D), jnp.float32),
            jax.ShapeDtypeStruct((NPAD, 1), jnp.float32),
            jax.ShapeDtypeStruct((NPAD, 1), jnp.float32),
        ],
    )(xpad, W1, doutp, dinp)


def _final_body(agg_ref, ri_ref, b1_ref, w2_ref, ro_ref, c_ref, b2_ref, o_ref):
    pid = pl.program_id(0)
    blk = agg_ref.shape[1]
    agg = agg_ref[0] + agg_ref[1]
    out1 = agg * ri_ref[...] + b1_ref[...]
    t = jnp.tanh(out1)
    w2s = jnp.sum(w2_ref[...], axis=1, keepdims=True)
    sblk = jnp.dot(t, w2s, preferred_element_type=jnp.float32)
    cc = c_ref[0] + c_ref[1]
    rowid = pid * blk + lax.broadcasted_iota(jnp.int32, (blk, 1), 0)
    valid = jnp.where(rowid < N_NODES, 1.0, 0.0)
    part = jnp.sum(ro_ref[...] * sblk * cc * valid, keepdims=True)

    @pl.when(pid == 0)
    def _():
        o_ref[...] = jnp.float32(N_NODES) * jnp.sum(b2_ref[...], keepdims=True)

    o_ref[...] += part


def _tc_final(aggp, rin, b1, W2, rout, cp, b2):
    blk = 1024
    return pl.pallas_call(
        _final_body,
        grid=(NPAD // blk,),
        in_specs=[
            pl.BlockSpec((NC, blk, D), lambda i: (0, i, 0)),
            pl.BlockSpec((blk, 1), lambda i: (i, 0)),
            pl.BlockSpec((1, D), lambda i: (0, 0)),
            pl.BlockSpec((D, D), lambda i: (0, 0)),
            pl.BlockSpec((blk, 1), lambda i: (i, 0)),
            pl.BlockSpec((NC, blk, 1), lambda i: (0, i, 0)),
            pl.BlockSpec((1, D), lambda i: (0, 0)),
        ],
        out_specs=pl.BlockSpec((1, 1), lambda i: (0, 0)),
        out_shape=jax.ShapeDtypeStruct((1, 1), jnp.float32),
    )(aggp, rin, b1, W2, rout, cp, b2)


def kernel(x, edge_index, W1, b1, W2, b2):
    ei = edge_index.astype(jnp.int32)
    src = ei[0]
    dst = ei[1]

    npad_e = EPAD - N_EDGES
    pad_idx = N_NODES + (jnp.arange(npad_e, dtype=jnp.int32) % PAD_ROWS)
    src_p = jnp.concatenate([src, pad_idx]).reshape(EPAD // CHUNK, CHUNK)
    dst_p = jnp.concatenate([dst, pad_idx]).reshape(EPAD // CHUNK, CHUNK)

    z2d = jnp.zeros((NPAD, D), jnp.float32)
    z1d = jnp.zeros((NPAD,), jnp.float32)

    doutp, dinp = _sc_degrees(src_p, dst_p, z1d)

    xpad = jnp.pad(x, ((0, NPAD - N_NODES), (0, 0)))
    hw2, rout, rin = _tc_fused(
        xpad, W1, doutp.reshape(NC, NPAD, 1), dinp.reshape(NC, NPAD, 1))

    aggp, cp = _sc_scatter(hw2, src_p, dst_p, rin.reshape(NPAD), z2d, z1d)

    out = _tc_final(aggp, rin, b1.reshape(1, D), W2, rout,
                    cp.reshape(NC, NPAD, 1), b2.reshape(1, D))
    return out


# speedup vs baseline: 1.1957x; 1.1164x over previous
import functools

import jax
import jax.numpy as jnp
from jax import lax
from jax.experimental import pallas as pl
from jax.experimental.pallas import tpu as pltpu
from jax.experimental.pallas import tpu_sc as plsc

N_NODES = 10000
D = 128
N_EDGES = 320000

NC = 2
NS = 16
NW = NC * NS

NPAD = 10240
ROWS_PER_TILE = NPAD // NS
PAD_ROWS = NPAD - N_NODES

CHUNK = 128
EPAD = 327680
EDGES_PER_W = EPAD // NW
NCHUNK = EDGES_PER_W // CHUNK

_MESH = plsc.VectorSubcoreMesh(core_axis_name="c", subcore_axis_name="s")


def _worker_ids():
    c = lax.axis_index("c")
    s = lax.axis_index("s")
    return c, s, c * NS + s


@functools.partial(
    pl.kernel,
    mesh=_MESH,
    out_type=[
        jax.ShapeDtypeStruct((NC, NPAD), jnp.float32),
        jax.ShapeDtypeStruct((NC, NPAD), jnp.float32),
    ],
    scratch_types=[
        pltpu.VMEM((NCHUNK, CHUNK), jnp.int32),
        pltpu.VMEM((NCHUNK, CHUNK), jnp.int32),
        pltpu.VMEM((CHUNK,), jnp.float32),
        pltpu.VMEM_SHARED((NPAD,), jnp.float32),
        pltpu.VMEM_SHARED((NPAD,), jnp.float32),
        pltpu.SemaphoreType.DMA,
    ],
)
def _sc_degrees(src_hbm, dst_hbm, z1d_hbm, dout_hbm, din_hbm,
                sidx, didx, ones, dout_sh, din_sh, ssem):
    c, s, wid = _worker_ids()

    def fill16(i, _):
        ones[pl.ds(i * 16, 16)] = jnp.full((16,), 1.0, jnp.float32)
        return 0
    lax.fori_loop(0, CHUNK // 16, fill16, 0)

    stripe = pl.ds(s * ROWS_PER_TILE, ROWS_PER_TILE)
    pltpu.sync_copy(z1d_hbm.at[stripe], dout_sh.at[stripe])
    pltpu.sync_copy(z1d_hbm.at[stripe], din_sh.at[stripe])

    pltpu.sync_copy(src_hbm.at[pl.ds(wid * NCHUNK, NCHUNK)], sidx)
    pltpu.sync_copy(dst_hbm.at[pl.ds(wid * NCHUNK, NCHUNK)], didx)

    plsc.subcore_barrier()

    def round_(r, _):
        for b in range(8):
            j = r * 8 + b
            pltpu.async_copy(ones, dout_sh.at[sidx.at[j]], ssem, add=True)
            pltpu.async_copy(ones, din_sh.at[didx.at[j]], ssem, add=True)
        for b in range(8):
            j = r * 8 + b
            pltpu.make_async_copy(ones, dout_sh.at[sidx.at[j]], ssem).wait()
            pltpu.make_async_copy(ones, din_sh.at[didx.at[j]], ssem).wait()
        return 0
    lax.fori_loop(0, NCHUNK // 8, round_, 0)

    plsc.subcore_barrier()

    pltpu.sync_copy(dout_sh.at[stripe], dout_hbm.at[c, stripe])
    pltpu.sync_copy(din_sh.at[stripe], din_hbm.at[c, stripe])


@functools.partial(
    pl.kernel,
    mesh=_MESH,
    out_type=[
        jax.ShapeDtypeStruct((NC, NPAD, D), jnp.float32),
        jax.ShapeDtypeStruct((NC, NPAD), jnp.float32),
    ],
    scratch_types=[
        pltpu.VMEM((NCHUNK // 2, CHUNK), jnp.int32),
        pltpu.VMEM((NCHUNK // 2, CHUNK), jnp.int32),
        pltpu.VMEM((2, CHUNK, D), jnp.float32),
        pltpu.VMEM((2, CHUNK), jnp.float32),
        pltpu.VMEM_SHARED((NPAD, D), jnp.float32),
        pltpu.VMEM_SHARED((NPAD,), jnp.float32),
        pltpu.SemaphoreType.DMA,
        pltpu.SemaphoreType.DMA,
        pltpu.SemaphoreType.DMA,
        pltpu.SemaphoreType.DMA,
        pltpu.SemaphoreType.DMA,
        pltpu.SemaphoreType.DMA,
    ],
)
def _sc_scatter(hw2_hbm, src_hbm, dst_hbm, rin_hbm, z2d_hbm, z1d_hbm,
                agg_hbm, c_hbm,
                sidx, didx, rows, rvals, agg_sh, c_sh,
                gs0, gs1, rv0, rv1, cs0, cs1):
    c, s, wid = _worker_ids()
    gs = (gs0, gs1)
    rv = (rv0, rv1)
    cs = (cs0, cs1)
    HALF = NCHUNK // 2

    stripe = pl.ds(s * ROWS_PER_TILE, ROWS_PER_TILE)
    pltpu.sync_copy(z2d_hbm.at[stripe], agg_sh.at[stripe])
    pltpu.sync_copy(z1d_hbm.at[stripe], c_sh.at[stripe])

    plsc.subcore_barrier()

    for h in range(2):
        pltpu.sync_copy(src_hbm.at[pl.ds(wid * NCHUNK + h * HALF, HALF)], sidx)
        pltpu.sync_copy(dst_hbm.at[pl.ds(wid * NCHUNK + h * HALF, HALF)], didx)

        for b in range(2):
            pltpu.async_copy(hw2_hbm.at[sidx.at[b]], rows.at[b], gs[b])
            pltpu.async_copy(rin_hbm.at[didx.at[b]], rvals.at[b], rv[b])

        def main(i, _):
            for b in range(2):
                j = i * 2 + b
                pltpu.make_async_copy(hw2_hbm.at[sidx.at[j]], rows.at[b],
                                      gs[b]).wait()
                pltpu.sync_copy(rows.at[b], agg_sh.at[didx.at[j]], add=True)

                @pl.when(j + 2 < HALF)
                def _():
                    pltpu.async_copy(hw2_hbm.at[sidx.at[j + 2]], rows.at[b],
                                     gs[b])

                pltpu.make_async_copy(rin_hbm.at[didx.at[j]], rvals.at[b],
                                      rv[b]).wait()
                pltpu.async_copy(rvals.at[b], c_sh.at[sidx.at[j]], cs[b],
                                 add=True)

                @pl.when(j + 2 < HALF)
                def _():
                    pltpu.make_async_copy(rvals.at[b], c_sh.at[sidx.at[j]],
                                          cs[b]).wait()
                    pltpu.async_copy(rin_hbm.at[didx.at[j + 2]], rvals.at[b],
                                     rv[b])
            return 0
        lax.fori_loop(0, HALF // 2, main, 0)

        for b in range(2):
            pltpu.make_async_copy(rvals.at[b], c_sh.at[sidx.at[0]],
                                  cs[b]).wait()

    plsc.subcore_barrier()

    pltpu.sync_copy(agg_sh.at[stripe], agg_hbm.at[c, stripe])
    pltpu.sync_copy(c_sh.at[stripe], c_hbm.at[c, stripe])


def _fuse_body(x_ref, w_ref, do_ref, di_ref, hw2_ref, ro_ref, ri_ref):
    do = do_ref[0] + do_ref[1]
    di = di_ref[0] + di_ref[1]
    ro = lax.rsqrt(jnp.maximum(do, 1.0))
    ri = lax.rsqrt(jnp.maximum(di, 1.0))
    hw2_ref[...] = jnp.dot(jnp.sin(x_ref[...]) * ro, w_ref[...],
                           preferred_element_type=jnp.float32)
    ro_ref[...] = ro
    ri_ref[...] = ri


def _tc_fused(xpad, W1, doutp, dinp):
    blk = 1024
    return pl.pallas_call(
        _fuse_body,
        grid=(NPAD // blk,),
        in_specs=[
            pl.BlockSpec((blk, D), lambda i: (i, 0)),
            pl.BlockSpec((D, D), lambda i: (0, 0)),
            pl.BlockSpec((NC, blk, 1), lambda i: (0, i, 0)),
            pl.BlockSpec((NC, blk, 1), lambda i: (0, i, 0)),
        ],
        out_specs=[
            pl.BlockSpec((blk, D), lambda i: (i, 0)),
            pl.BlockSpec((blk, 1), lambda i: (i, 0)),
            pl.BlockSpec((blk, 1), lambda i: (i, 0)),
        ],
        out_shape=[
            jax.ShapeDtypeStruct((NPAD, D), jnp.float32),
            jax.ShapeDtypeStruct((NPAD, 1), jnp.float32),
            jax.ShapeDtypeStruct((NPAD, 1), jnp.float32),
        ],
    )(xpad, W1, doutp, dinp)


def _final_body(agg_ref, ri_ref, b1_ref, w2_ref, ro_ref, c_ref, b2_ref, o_ref):
    pid = pl.program_id(0)
    blk = agg_ref.shape[1]
    agg = agg_ref[0] + agg_ref[1]
    out1 = agg * ri_ref[...] + b1_ref[...]
    t = jnp.tanh(out1)
    w2s = jnp.sum(w2_ref[...], axis=1, keepdims=True)
    sblk = jnp.dot(t, w2s, preferred_element_type=jnp.float32)
    cc = c_ref[0] + c_ref[1]
    rowid = pid * blk + lax.broadcasted_iota(jnp.int32, (blk, 1), 0)
    valid = jnp.where(rowid < N_NODES, 1.0, 0.0)
    part = jnp.sum(ro_ref[...] * sblk * cc * valid, keepdims=True)

    @pl.when(pid == 0)
    def _():
        o_ref[...] = jnp.float32(N_NODES) * jnp.sum(b2_ref[...], keepdims=True)

    o_ref[...] += part


def _tc_final(aggp, rin, b1, W2, rout, cp, b2):
    blk = 1024
    return pl.pallas_call(
        _final_body,
        grid=(NPAD // blk,),
        in_specs=[
            pl.BlockSpec((NC, blk, D), lambda i: (0, i, 0)),
            pl.BlockSpec((blk, 1), lambda i: (i, 0)),
            pl.BlockSpec((1, D), lambda i: (0, 0)),
            pl.BlockSpec((D, D), lambda i: (0, 0)),
            pl.BlockSpec((blk, 1), lambda i: (i, 0)),
            pl.BlockSpec((NC, blk, 1), lambda i: (0, i, 0)),
            pl.BlockSpec((1, D), lambda i: (0, 0)),
        ],
        out_specs=pl.BlockSpec((1, 1), lambda i: (0, 0)),
        out_shape=jax.ShapeDtypeStruct((1, 1), jnp.float32),
    )(aggp, rin, b1, W2, rout, cp, b2)


def kernel(x, edge_index, W1, b1, W2, b2):
    ei = edge_index.astype(jnp.int32)
    src = ei[0]
    dst = ei[1]

    npad_e = EPAD - N_EDGES
    pad_idx = N_NODES + (jnp.arange(npad_e, dtype=jnp.int32) % PAD_ROWS)
    src_p = jnp.concatenate([src, pad_idx]).reshape(EPAD // CHUNK, CHUNK)
    dst_p = jnp.concatenate([dst, pad_idx]).reshape(EPAD // CHUNK, CHUNK)

    z2d = jnp.zeros((NPAD, D), jnp.float32)
    z1d = jnp.zeros((NPAD,), jnp.float32)

    doutp, dinp = _sc_degrees(src_p, dst_p, z1d)

    xpad = jnp.pad(x, ((0, NPAD - N_NODES), (0, 0)))
    hw2, rout, rin = _tc_fused(
        xpad, W1, doutp.reshape(NC, NPAD, 1), dinp.reshape(NC, NPAD, 1))

    aggp, cp = _sc_scatter(hw2, src_p, dst_p, rin.reshape(NPAD), z2d, z1d)

    out = _tc_final(aggp, rin, b1.reshape(1, D), W2, rout,
                    cp.reshape(NC, NPAD, 1), b2.reshape(1, D))
    return out
